# Initial kernel scaffold; baseline (speedup 1.0000x reference)
import jax
import jax.numpy as jnp
from jax.experimental import pallas as pl


def kernel(x, W1, b1, w2, Wo, bo, gamma, beta):
    raise NotImplementedError("write your pallas kernel here")


# speedup vs baseline: 3.1126x; 3.1126x over previous
import functools

import jax
import jax.numpy as jnp
from jax.experimental import pallas as pl
from jax.experimental.pallas import tpu as pltpu

_EPS = 1e-5
_CHUNK = 512


def _body(x_ref, w1_ref, b1_ref, st_ref, wo_ref, bo_ref, g_ref, be_ref,
          out_ref, w_out_ref, *, t, h_heads):
    w1 = w1_ref[...]
    b1 = b1_ref[...]
    st = st_ref[...]

    parts = []
    for i in range(t // _CHUNK):
        xc = x_ref[0, i * _CHUNK:(i + 1) * _CHUNK, :]
        hc = jnp.tanh(jnp.dot(xc, w1, preferred_element_type=jnp.float32) + b1)
        parts.append(jax.lax.dot_general(
            st, hc, (((1,), (1,)), ((), ())),
            preferred_element_type=jnp.float32))
    scores = jnp.concatenate(parts, axis=1)

    m = jnp.max(scores, axis=1, keepdims=True)
    e = jnp.exp(scores - m)
    denom = jnp.sum(e, axis=1, keepdims=True)
    w = e / denom
    w_out_ref[0] = w

    ctx = jnp.zeros((h_heads, x_ref.shape[2]), dtype=jnp.float32)
    for i in range(t // _CHUNK):
        xc = x_ref[0, i * _CHUNK:(i + 1) * _CHUNK, :]
        ctx = ctx + jnp.dot(w[:, i * _CHUNK:(i + 1) * _CHUNK], xc,
                            preferred_element_type=jnp.float32)

    acc = bo_ref[...]
    for h in range(h_heads):
        acc = acc + jnp.dot(ctx[h:h + 1, :], wo_ref[h],
                            preferred_element_type=jnp.float32)

    mu = jnp.mean(acc, axis=1, keepdims=True)
    d = acc - mu
    var = jnp.mean(d * d, axis=1, keepdims=True)
    out_ref[0] = d * jax.lax.rsqrt(var + _EPS) * g_ref[...] + be_ref[...]


def kernel(x, W1, b1, w2, Wo, bo, gamma, beta):
    b, t, c = x.shape
    h_heads, _, hd = W1.shape

    w1c = jnp.transpose(W1, (1, 0, 2)).reshape(c, h_heads * hd)
    b1f = b1.reshape(1, h_heads * hd)
    st = (jnp.eye(h_heads, dtype=x.dtype)[:, :, None] * w2[None, :, :]
          ).reshape(h_heads, h_heads * hd)
    wo_h = jnp.transpose(Wo.reshape(c, h_heads, c), (1, 2, 0))

    body = functools.partial(_body, t=t, h_heads=h_heads)
    out3, wts = pl.pallas_call(
        body,
        grid=(b,),
        in_specs=[
            pl.BlockSpec((1, t, c), lambda i: (i, 0, 0)),
            pl.BlockSpec((c, h_heads * hd), lambda i: (0, 0)),
            pl.BlockSpec((1, h_heads * hd), lambda i: (0, 0)),
            pl.BlockSpec((h_heads, h_heads * hd), lambda i: (0, 0)),
            pl.BlockSpec((h_heads, c, c), lambda i: (0, 0, 0)),
            pl.BlockSpec((1, c), lambda i: (0, 0)),
            pl.BlockSpec((1, c), lambda i: (0, 0)),
            pl.BlockSpec((1, c), lambda i: (0, 0)),
        ],
        out_specs=[
            pl.BlockSpec((1, 1, c), lambda i: (i, 0, 0)),
            pl.BlockSpec((1, h_heads, t), lambda i: (i, 0, 0)),
        ],
        out_shape=[
            jax.ShapeDtypeStruct((b, 1, c), jnp.float32),
            jax.ShapeDtypeStruct((b, h_heads, t), jnp.float32),
        ],
        compiler_params=pltpu.CompilerParams(
            dimension_semantics=("parallel",),
        ),
    )(x, w1c, b1f, st, wo_h, bo.reshape(1, c), gamma.reshape(1, c),
      beta.reshape(1, c))
    return out3.reshape(b, c), wts
